# Initial kernel scaffold; baseline (speedup 1.0000x reference)
import jax
import jax.numpy as jnp
from jax.experimental import pallas as pl


def kernel(inp, pos, neg, emb, emb_f, emb_r):
    raise NotImplementedError("write your pallas kernel here")


# speedup vs baseline: 3.5819x; 3.5819x over previous
import functools

import jax
import jax.numpy as jnp
from jax import lax
from jax.experimental import pallas as pl
from jax.experimental.pallas import tpu as pltpu
from jax.experimental.pallas import tpu_sc as plsc

B = 16384
D = 64
K = 28
NC = 2
NS = 16
NW = NC * NS
RW = B // NW
C = 16
NCH = RW // C
KC = C * K
G = 4
GS = KC // G


def _sc_body(ctx_hbm, inp6_hbm, inp0_hbm, emb_hbm, embf_hbm, embr_hbm,
             pred_hbm, idx_r, idx_e, idx_f, r_buf, e_buf, f_buf, pred_buf,
             stage, sem):
    wid = lax.axis_index("s") * NC + lax.axis_index("c")

    def chunk_body(c, carry):
        ch = wid * NCH + c

        pltpu.sync_copy(ctx_hbm.at[pl.ds(ch * G, G)], idx_r)
        pltpu.sync_copy(inp6_hbm.at[pl.ds(ch, 1)], idx_e)
        pltpu.sync_copy(inp0_hbm.at[pl.ds(ch, 1)], idx_f)

        hs = [
            pltpu.async_copy(embr_hbm.at[idx_r.at[g]],
                             r_buf.at[pl.ds(g * GS, GS)], sem)
            for g in range(G)
        ]
        hs.append(pltpu.async_copy(emb_hbm.at[idx_e.at[0]], e_buf, sem))
        hs.append(pltpu.async_copy(embf_hbm.at[idx_f.at[0]], f_buf, sem))
        for h in hs:
            h.wait()

        def row_body(i, carry2):
            f = [f_buf[i, pl.ds(s * 16, 16)] for s in range(8)]

            def e(j, s):
                return e_buf[i * 6 + j, pl.ds(s * 16, 16)]

            v = []
            for s in range(4):
                v.append((f[s] + e(0, s) + e(3, s)) * (1.0 / 3.0))
            for s in range(4):
                v.append((f[4 + s]
                          + (e(1, s) + e(2, s)) * 0.5
                          + (e(4, s) + e(5, s)) * 0.5) * (1.0 / 3.0))

            for k in range(K):
                row = i * K + k
                acc = r_buf[row, pl.ds(0, 16)] * v[0]
                for s in range(1, 8):
                    acc = acc + r_buf[row, pl.ds(s * 16, 16)] * v[s]
                for sh in (1, 2, 4, 8):
                    perm = jnp.arange(16, dtype=jnp.int32)
                    perm = (perm + sh) % 16
                    acc = acc + acc.at[perm].get(mode="promise_in_bounds")
                stage[pl.ds(row * 16, 16)] = acc
            return carry2

        lax.fori_loop(0, C, row_body, 0)

        lanes = jnp.arange(16, dtype=jnp.int32) * 16 + 15
        for g in range(KC // 16):
            vec = plsc.load_gather(stage, [lanes + g * 256])
            pred_buf[pl.ds(g * 16, 16)] = vec

        pltpu.sync_copy(pred_buf, pred_hbm.at[pl.ds(ch * KC, KC)])
        return carry

    lax.fori_loop(0, NCH, chunk_body, 0)


_sc_pred = functools.partial(
    pl.kernel,
    out_type=jax.ShapeDtypeStruct((B * K,), jnp.float32),
    mesh=plsc.VectorSubcoreMesh(core_axis_name="c", subcore_axis_name="s"),
    compiler_params=pltpu.CompilerParams(needs_layout_passes=False,
                                         use_tc_tiling_on_sc=False),
    scratch_types=[
        pltpu.VMEM((G, GS), jnp.int32),
        pltpu.VMEM((1, C * 6), jnp.int32),
        pltpu.VMEM((1, C), jnp.int32),
        pltpu.VMEM((KC, 2 * D), jnp.float32),
        pltpu.VMEM((C * 6, D), jnp.float32),
        pltpu.VMEM((C, 2 * D), jnp.float32),
        pltpu.VMEM((KC,), jnp.float32),
        pltpu.VMEM((KC * 16,), jnp.float32),
        pltpu.SemaphoreType.DMA,
    ],
)(_sc_body)


def _bce_body(pred_ref, out_ref):
    x = pred_ref[...]
    rows = lax.broadcasted_iota(jnp.int32, x.shape, 0)
    cols = lax.broadcasted_iota(jnp.int32, x.shape, 1)
    flat = rows * x.shape[1] + cols
    label = (flat % K) < 3
    p = jax.nn.sigmoid(x)
    p = jnp.clip(p, 1e-7, 1.0 - 1e-7)
    term = jnp.where(label, jnp.log(p), jnp.log(1.0 - p))
    loss = -jnp.sum(term) * (1.0 / (B * K))
    out_ref[...] = jnp.reshape(loss, (1, 1))


def kernel(inp, pos, neg, emb, emb_f, emb_r):
    ctx = jnp.concatenate([pos.astype(jnp.int32), neg.astype(jnp.int32)],
                          axis=1)
    ctx2 = ctx.reshape(B * K // GS, GS)
    inp6 = inp[:, 1:].astype(jnp.int32).reshape(B // C, C * 6)
    inp0 = inp[:, 0].astype(jnp.int32).reshape(B // C, C)

    pred = _sc_pred(ctx2, inp6, inp0, emb, emb_f, emb_r)

    n_rows = B * K // 128
    loss = pl.pallas_call(
        _bce_body,
        out_shape=jax.ShapeDtypeStruct((1, 1), jnp.float32),
    )(pred.reshape(n_rows, 128))
    return jnp.reshape(loss, ())


# speedup vs baseline: 4.6431x; 1.2963x over previous
import functools

import jax
import jax.numpy as jnp
from jax import lax
from jax.experimental import pallas as pl
from jax.experimental.pallas import tpu as pltpu
from jax.experimental.pallas import tpu_sc as plsc

B = 16384
D = 64
K = 28
NC = 2
NS = 16
NW = NC * NS
RW = B // NW
C = 8
NCH = RW // C
KC = C * K
G = 2
GS = KC // G


def _sc_body(ctx_hbm, ief_hbm, emb_hbm, embf_hbm, embr_hbm, pred_hbm,
             idx_ctx, idx_ef, r_a, r_b, e_a, e_b, f_a, f_b, pred_a, pred_b,
             stage, sem_a, sem_b, psem_a, psem_b):
    wid = lax.axis_index("s") * NC + lax.axis_index("c")

    pltpu.sync_copy(ctx_hbm.at[pl.ds(wid * (NCH * G), NCH * G)], idx_ctx)
    pltpu.sync_copy(ief_hbm.at[pl.ds(wid * NCH, NCH)], idx_ef)

    def fire(c, r_buf, e_buf, f_buf, sem):
        for g in range(G):
            pltpu.async_copy(embr_hbm.at[idx_ctx.at[c * G + g]],
                             r_buf.at[pl.ds(g * GS, GS)], sem)
        pltpu.async_copy(embf_hbm.at[idx_ef.at[c, pl.ds(0, C)]], f_buf, sem)
        pltpu.async_copy(emb_hbm.at[idx_ef.at[c, pl.ds(C, 6 * C)]], e_buf,
                         sem)

    def drain(r_buf, e_buf, f_buf, sem):
        for g in range(G):
            pltpu.make_async_copy(embr_hbm.at[pl.ds(0, GS)],
                                  r_buf.at[pl.ds(g * GS, GS)], sem).wait()
        pltpu.make_async_copy(embf_hbm.at[pl.ds(0, C)], f_buf, sem).wait()
        pltpu.make_async_copy(emb_hbm.at[pl.ds(0, 6 * C)], e_buf,
                              sem).wait()

    def compute(r_buf, e_buf, f_buf, pred_buf):
        def row_body(i, carry):
            f = [f_buf[i, pl.ds(s * 16, 16)] for s in range(8)]

            def e(j, s):
                return e_buf[i * 6 + j, pl.ds(s * 16, 16)]

            v = []
            for s in range(4):
                v.append((f[s] + e(0, s) + e(3, s)) * (1.0 / 3.0))
            for s in range(4):
                v.append((f[4 + s]
                          + (e(1, s) + e(2, s)) * 0.5
                          + (e(4, s) + e(5, s)) * 0.5) * (1.0 / 3.0))

            for k in range(K):
                row = i * K + k
                acc = r_buf[row, pl.ds(0, 16)] * v[0]
                for s in range(1, 8):
                    acc = acc + r_buf[row, pl.ds(s * 16, 16)] * v[s]
                for sh in (1, 2, 4, 8):
                    perm = (jnp.arange(16, dtype=jnp.int32) + sh) % 16
                    acc = acc + acc.at[perm].get(mode="promise_in_bounds")
                stage[pl.ds(row * 16, 16)] = acc
            return carry

        lax.fori_loop(0, C, row_body, 0)

        lanes = jnp.arange(16, dtype=jnp.int32) * 16 + 15
        for g in range(KC // 16):
            vec = plsc.load_gather(stage, [lanes + g * 256])
            pred_buf[pl.ds(g * 16, 16)] = vec

    def pred_fire(ch, pred_buf, psem):
        pltpu.async_copy(pred_buf, pred_hbm.at[pl.ds(ch * KC, KC)], psem)

    def pred_drain(pred_buf, psem):
        pltpu.make_async_copy(pred_buf, pred_hbm.at[pl.ds(0, KC)],
                              psem).wait()

    fire(0, r_a, e_a, f_a, sem_a)
    npair = NCH // 2

    def pair(j, carry):
        c0 = j * 2
        ch0 = wid * NCH + c0

        fire(c0 + 1, r_b, e_b, f_b, sem_b)
        drain(r_a, e_a, f_a, sem_a)

        @pl.when(j > 0)
        def _():
            pred_drain(pred_a, psem_a)

        compute(r_a, e_a, f_a, pred_a)
        pred_fire(ch0, pred_a, psem_a)

        @pl.when(j < npair - 1)
        def _():
            fire(c0 + 2, r_a, e_a, f_a, sem_a)

        drain(r_b, e_b, f_b, sem_b)

        @pl.when(j > 0)
        def _():
            pred_drain(pred_b, psem_b)

        compute(r_b, e_b, f_b, pred_b)
        pred_fire(ch0 + 1, pred_b, psem_b)
        return carry

    lax.fori_loop(0, npair, pair, 0)
    pred_drain(pred_a, psem_a)
    pred_drain(pred_b, psem_b)


_sc_pred = functools.partial(
    pl.kernel,
    out_type=jax.ShapeDtypeStruct((B * K,), jnp.float32),
    mesh=plsc.VectorSubcoreMesh(core_axis_name="c", subcore_axis_name="s"),
    compiler_params=pltpu.CompilerParams(needs_layout_passes=False,
                                         use_tc_tiling_on_sc=False),
    scratch_types=[
        pltpu.VMEM((NCH * G, GS), jnp.int32),
        pltpu.VMEM((NCH, 7 * C), jnp.int32),
        pltpu.VMEM((KC, 2 * D), jnp.float32),
        pltpu.VMEM((KC, 2 * D), jnp.float32),
        pltpu.VMEM((C * 6, D), jnp.float32),
        pltpu.VMEM((C * 6, D), jnp.float32),
        pltpu.VMEM((C, 2 * D), jnp.float32),
        pltpu.VMEM((C, 2 * D), jnp.float32),
        pltpu.VMEM((KC,), jnp.float32),
        pltpu.VMEM((KC,), jnp.float32),
        pltpu.VMEM((KC * 16,), jnp.float32),
        pltpu.SemaphoreType.DMA,
        pltpu.SemaphoreType.DMA,
        pltpu.SemaphoreType.DMA,
        pltpu.SemaphoreType.DMA,
    ],
)(_sc_body)


def _bce_body(pred_ref, out_ref):
    x = pred_ref[...]
    rows = lax.broadcasted_iota(jnp.int32, x.shape, 0)
    cols = lax.broadcasted_iota(jnp.int32, x.shape, 1)
    flat = rows * x.shape[1] + cols
    label = (flat % K) < 3
    p = jax.nn.sigmoid(x)
    p = jnp.clip(p, 1e-7, 1.0 - 1e-7)
    term = jnp.where(label, jnp.log(p), jnp.log(1.0 - p))
    loss = -jnp.sum(term) * (1.0 / (B * K))
    out_ref[...] = jnp.reshape(loss, (1, 1))


def kernel(inp, pos, neg, emb, emb_f, emb_r):
    ctx = jnp.concatenate([pos.astype(jnp.int32), neg.astype(jnp.int32)],
                          axis=1)
    ctx2 = ctx.reshape(B * K // GS, GS)
    inp32 = inp.astype(jnp.int32)
    ief = jnp.concatenate([inp32[:, 0].reshape(B // C, C),
                           inp32[:, 1:].reshape(B // C, C * 6)], axis=1)

    pred = _sc_pred(ctx2, ief, emb, emb_f, emb_r)

    n_rows = B * K // 128
    loss = pl.pallas_call(
        _bce_body,
        out_shape=jax.ShapeDtypeStruct((1, 1), jnp.float32),
    )(pred.reshape(n_rows, 128))
    return jnp.reshape(loss, ())


# speedup vs baseline: 9.5044x; 2.0470x over previous
import functools

import jax
import jax.numpy as jnp
from jax import lax
from jax.experimental import pallas as pl
from jax.experimental.pallas import tpu as pltpu
from jax.experimental.pallas import tpu_sc as plsc

B = 16384
D = 64
K = 28
KP = 32
NC = 2
NS = 16
NW = NC * NS
RW = B // NW
C = 8
NCH = RW // C
KC = C * K
G = 2
GS = KC // G


def _sc_body(ctx_hbm, ief_hbm, emb_hbm, embf_hbm, embr_hbm, pred_hbm,
             idx_ctx, idx_ef, r_a, r_b, e_a, e_b, f_a, f_b, pred_a, pred_b,
             sem_a, sem_b, psem_a, psem_b):
    wid = lax.axis_index("s") * NC + lax.axis_index("c")

    pltpu.sync_copy(ctx_hbm.at[pl.ds(wid * (NCH * G), NCH * G)], idx_ctx)
    pltpu.sync_copy(ief_hbm.at[pl.ds(wid * NCH, NCH)], idx_ef)

    def fire(c, r_buf, e_buf, f_buf, sem):
        for g in range(G):
            pltpu.async_copy(embr_hbm.at[idx_ctx.at[c * G + g]],
                             r_buf.at[pl.ds(g * GS, GS)], sem)
        pltpu.async_copy(embf_hbm.at[idx_ef.at[c, pl.ds(0, C)]], f_buf, sem)
        pltpu.async_copy(emb_hbm.at[idx_ef.at[c, pl.ds(C, 6 * C)]], e_buf,
                         sem)

    def drain(r_buf, e_buf, f_buf, sem):
        for g in range(G):
            pltpu.make_async_copy(embr_hbm.at[pl.ds(0, GS)],
                                  r_buf.at[pl.ds(g * GS, GS)], sem).wait()
        pltpu.make_async_copy(embf_hbm.at[pl.ds(0, C)], f_buf, sem).wait()
        pltpu.make_async_copy(emb_hbm.at[pl.ds(0, 6 * C)], e_buf,
                              sem).wait()

    lane = jnp.arange(16, dtype=jnp.int32)
    perms = [(lane + sh) % 16 for sh in (1, 2, 4, 8)]
    zeros = jnp.zeros((16,), jnp.float32)

    def compute(r_buf, e_buf, f_buf, pred_buf):
        def row_body(i, carry):
            f = [f_buf[i, pl.ds(s * 16, 16)] for s in range(8)]

            def e(j, s):
                return e_buf[i * 6 + j, pl.ds(s * 16, 16)]

            v = []
            for s in range(4):
                v.append((f[s] + e(0, s) + e(3, s)) * (1.0 / 3.0))
            for s in range(4):
                v.append((f[4 + s]
                          + (e(1, s) + e(2, s)) * 0.5
                          + (e(4, s) + e(5, s)) * 0.5) * (1.0 / 3.0))

            out = [zeros, zeros]
            for k in range(K):
                row = i * K + k
                p = [r_buf[row, pl.ds(s * 16, 16)] * v[s] for s in range(8)]
                while len(p) > 1:
                    p = [p[2 * t] + p[2 * t + 1] for t in range(len(p) // 2)]
                acc = p[0]
                for perm in perms:
                    acc = acc + acc.at[perm].get(mode="promise_in_bounds")
                half, ph = divmod(k, 16)
                out[half] = jnp.where(lane == ph, acc, out[half])
            pred_buf[i, pl.ds(0, 16)] = out[0]
            pred_buf[i, pl.ds(16, 16)] = out[1]
            return carry

        lax.fori_loop(0, C, row_body, 0)

    def pred_fire(ch, pred_buf, psem):
        pltpu.async_copy(pred_buf, pred_hbm.at[pl.ds(ch * C, C)], psem)

    def pred_drain(pred_buf, psem):
        pltpu.make_async_copy(pred_buf, pred_hbm.at[pl.ds(0, C)],
                              psem).wait()

    fire(0, r_a, e_a, f_a, sem_a)
    npair = NCH // 2

    def pair(j, carry):
        c0 = j * 2
        ch0 = wid * NCH + c0

        fire(c0 + 1, r_b, e_b, f_b, sem_b)
        drain(r_a, e_a, f_a, sem_a)

        @pl.when(j > 0)
        def _():
            pred_drain(pred_a, psem_a)

        compute(r_a, e_a, f_a, pred_a)
        pred_fire(ch0, pred_a, psem_a)

        @pl.when(j < npair - 1)
        def _():
            fire(c0 + 2, r_a, e_a, f_a, sem_a)

        drain(r_b, e_b, f_b, sem_b)

        @pl.when(j > 0)
        def _():
            pred_drain(pred_b, psem_b)

        compute(r_b, e_b, f_b, pred_b)
        pred_fire(ch0 + 1, pred_b, psem_b)
        return carry

    lax.fori_loop(0, npair, pair, 0)
    pred_drain(pred_a, psem_a)
    pred_drain(pred_b, psem_b)


_sc_pred = functools.partial(
    pl.kernel,
    out_type=jax.ShapeDtypeStruct((B, KP), jnp.float32),
    mesh=plsc.VectorSubcoreMesh(core_axis_name="c", subcore_axis_name="s"),
    compiler_params=pltpu.CompilerParams(needs_layout_passes=False,
                                         use_tc_tiling_on_sc=False),
    scratch_types=[
        pltpu.VMEM((NCH * G, GS), jnp.int32),
        pltpu.VMEM((NCH, 7 * C), jnp.int32),
        pltpu.VMEM((KC, 2 * D), jnp.float32),
        pltpu.VMEM((KC, 2 * D), jnp.float32),
        pltpu.VMEM((C * 6, D), jnp.float32),
        pltpu.VMEM((C * 6, D), jnp.float32),
        pltpu.VMEM((C, 2 * D), jnp.float32),
        pltpu.VMEM((C, 2 * D), jnp.float32),
        pltpu.VMEM((C, KP), jnp.float32),
        pltpu.VMEM((C, KP), jnp.float32),
        pltpu.SemaphoreType.DMA,
        pltpu.SemaphoreType.DMA,
        pltpu.SemaphoreType.DMA,
        pltpu.SemaphoreType.DMA,
    ],
)(_sc_body)


def _bce_body(pred_ref, out_ref):
    x = pred_ref[...]
    rows = lax.broadcasted_iota(jnp.int32, x.shape, 0)
    cols = lax.broadcasted_iota(jnp.int32, x.shape, 1)
    k = (rows * x.shape[1] + cols) % KP
    label = k < 3
    valid = k < K
    p = jax.nn.sigmoid(x)
    p = jnp.clip(p, 1e-7, 1.0 - 1e-7)
    term = jnp.where(label, jnp.log(p), jnp.log(1.0 - p))
    term = jnp.where(valid, term, 0.0)
    loss = -jnp.sum(term) * (1.0 / (B * K))
    out_ref[...] = jnp.reshape(loss, (1, 1))


def kernel(inp, pos, neg, emb, emb_f, emb_r):
    ctx = jnp.concatenate([pos.astype(jnp.int32), neg.astype(jnp.int32)],
                          axis=1)
    ctx2 = ctx.reshape(B * K // GS, GS)
    inp32 = inp.astype(jnp.int32)
    ief = jnp.concatenate([inp32[:, 0].reshape(B // C, C),
                           inp32[:, 1:].reshape(B // C, C * 6)], axis=1)

    pred = _sc_pred(ctx2, ief, emb, emb_f, emb_r)

    n_rows = B * KP // 128
    loss = pl.pallas_call(
        _bce_body,
        out_shape=jax.ShapeDtypeStruct((1, 1), jnp.float32),
    )(pred.reshape(n_rows, 128))
    return jnp.reshape(loss, ())


# speedup vs baseline: 10.1632x; 1.0693x over previous
import functools

import jax
import jax.numpy as jnp
from jax import lax
from jax.experimental import pallas as pl
from jax.experimental.pallas import tpu as pltpu
from jax.experimental.pallas import tpu_sc as plsc

B = 16384
D = 64
K = 28
KP = 32
NC = 2
NS = 16
NW = NC * NS
RW = B // NW
C = 8
NCH = RW // C
KC = C * K
G = 2
GS = KC // G
NBLK = RW // 16


def _sc_body(ctxt_hbm, inpt_hbm, emb_hbm, embf_hbm, embr_hbm, pred_hbm,
             sta_ctx, sta_inp, idx_lin, e_lin, r_a, r_b, e_a, e_b, f_a,
             f_b, pred_a, pred_b, sem_a, sem_b, psem_a, psem_b):
    wid = lax.axis_index("s") * NC + lax.axis_index("c")
    b0 = wid * RW

    pltpu.sync_copy(ctxt_hbm.at[:, pl.ds(b0, RW)], sta_ctx)
    pltpu.sync_copy(inpt_hbm.at[:, pl.ds(b0, RW)], sta_inp)

    lane = jnp.arange(16, dtype=jnp.int32)
    ctx_dst = lane * K
    e_dst = lane * 6

    def tr_body(blk, carry):
        for k in range(K):
            vec = sta_ctx[k, pl.ds(blk * 16, 16)]
            plsc.store_scatter(idx_lin, [ctx_dst + (blk * (16 * K) + k)],
                               vec)
        for j in range(6):
            vec = sta_inp[1 + j, pl.ds(blk * 16, 16)]
            plsc.store_scatter(e_lin, [e_dst + (blk * 96 + j)], vec)
        return carry

    lax.fori_loop(0, NBLK, tr_body, 0)

    def fire(c, r_buf, e_buf, f_buf, sem):
        for g in range(G):
            pltpu.async_copy(
                embr_hbm.at[idx_lin.at[pl.ds(c * KC + g * GS, GS)]],
                r_buf.at[pl.ds(g * GS, GS)], sem)
        pltpu.async_copy(embf_hbm.at[sta_inp.at[0, pl.ds(c * C, C)]],
                         f_buf, sem)
        pltpu.async_copy(emb_hbm.at[e_lin.at[pl.ds(c * 6 * C, 6 * C)]],
                         e_buf, sem)

    def drain(r_buf, e_buf, f_buf, sem):
        for g in range(G):
            pltpu.make_async_copy(embr_hbm.at[pl.ds(0, GS)],
                                  r_buf.at[pl.ds(g * GS, GS)], sem).wait()
        pltpu.make_async_copy(embf_hbm.at[pl.ds(0, C)], f_buf, sem).wait()
        pltpu.make_async_copy(emb_hbm.at[pl.ds(0, 6 * C)], e_buf,
                              sem).wait()

    perms = [(lane + sh) % 16 for sh in (1, 2, 4, 8)]
    zeros = jnp.zeros((16,), jnp.float32)

    def compute(r_buf, e_buf, f_buf, pred_buf):
        def row_body(i, carry):
            f = [f_buf[i, pl.ds(s * 16, 16)] for s in range(8)]

            def e(j, s):
                return e_buf[i * 6 + j, pl.ds(s * 16, 16)]

            v = []
            for s in range(4):
                v.append((f[s] + e(0, s) + e(3, s)) * (1.0 / 3.0))
            for s in range(4):
                v.append((f[4 + s]
                          + (e(1, s) + e(2, s)) * 0.5
                          + (e(4, s) + e(5, s)) * 0.5) * (1.0 / 3.0))

            out = [zeros, zeros]
            for k in range(K):
                row = i * K + k
                p = [r_buf[row, pl.ds(s * 16, 16)] * v[s] for s in range(8)]
                while len(p) > 1:
                    p = [p[2 * t] + p[2 * t + 1] for t in range(len(p) // 2)]
                acc = p[0]
                for perm in perms:
                    acc = acc + acc.at[perm].get(mode="promise_in_bounds")
                half, ph = divmod(k, 16)
                out[half] = jnp.where(lane == ph, acc, out[half])
            pred_buf[pl.ds(i * KP, 16)] = out[0]
            pred_buf[pl.ds(i * KP + 16, 16)] = out[1]
            return carry

        lax.fori_loop(0, C, row_body, 0)

    def pred_fire(ch, pred_buf, psem):
        pltpu.async_copy(pred_buf, pred_hbm.at[pl.ds(ch * C * KP, C * KP)],
                         psem)

    def pred_drain(pred_buf, psem):
        pltpu.make_async_copy(pred_buf, pred_hbm.at[pl.ds(0, C * KP)],
                              psem).wait()

    fire(0, r_a, e_a, f_a, sem_a)
    npair = NCH // 2

    def pair(j, carry):
        c0 = j * 2
        ch0 = wid * NCH + c0

        fire(c0 + 1, r_b, e_b, f_b, sem_b)
        drain(r_a, e_a, f_a, sem_a)

        @pl.when(j > 0)
        def _():
            pred_drain(pred_a, psem_a)

        compute(r_a, e_a, f_a, pred_a)
        pred_fire(ch0, pred_a, psem_a)

        @pl.when(j < npair - 1)
        def _():
            fire(c0 + 2, r_a, e_a, f_a, sem_a)

        drain(r_b, e_b, f_b, sem_b)

        @pl.when(j > 0)
        def _():
            pred_drain(pred_b, psem_b)

        compute(r_b, e_b, f_b, pred_b)
        pred_fire(ch0 + 1, pred_b, psem_b)
        return carry

    lax.fori_loop(0, npair, pair, 0)
    pred_drain(pred_a, psem_a)
    pred_drain(pred_b, psem_b)


_sc_pred = functools.partial(
    pl.kernel,
    out_type=jax.ShapeDtypeStruct((B * KP,), jnp.float32),
    mesh=plsc.VectorSubcoreMesh(core_axis_name="c", subcore_axis_name="s"),
    compiler_params=pltpu.CompilerParams(needs_layout_passes=False,
                                         use_tc_tiling_on_sc=False),
    scratch_types=[
        pltpu.VMEM((K, RW), jnp.int32),
        pltpu.VMEM((7, RW), jnp.int32),
        pltpu.VMEM((NCH * KC,), jnp.int32),
        pltpu.VMEM((NCH * 6 * C,), jnp.int32),
        pltpu.VMEM((KC, 2 * D), jnp.float32),
        pltpu.VMEM((KC, 2 * D), jnp.float32),
        pltpu.VMEM((C * 6, D), jnp.float32),
        pltpu.VMEM((C * 6, D), jnp.float32),
        pltpu.VMEM((C, 2 * D), jnp.float32),
        pltpu.VMEM((C, 2 * D), jnp.float32),
        pltpu.VMEM((C * KP,), jnp.float32),
        pltpu.VMEM((C * KP,), jnp.float32),
        pltpu.SemaphoreType.DMA,
        pltpu.SemaphoreType.DMA,
        pltpu.SemaphoreType.DMA,
        pltpu.SemaphoreType.DMA,
    ],
)(_sc_body)


def _bce_body(pred_ref, out_ref):
    x = pred_ref[...]
    rows = lax.broadcasted_iota(jnp.int32, x.shape, 0)
    cols = lax.broadcasted_iota(jnp.int32, x.shape, 1)
    k = (rows * x.shape[1] + cols) % KP
    label = k < 3
    valid = k < K
    p = jax.nn.sigmoid(x)
    p = jnp.clip(p, 1e-7, 1.0 - 1e-7)
    term = jnp.where(label, jnp.log(p), jnp.log(1.0 - p))
    term = jnp.where(valid, term, 0.0)
    loss = -jnp.sum(term) * (1.0 / (B * K))
    out_ref[...] = jnp.reshape(loss, (1, 1))


def kernel(inp, pos, neg, emb, emb_f, emb_r):
    ctxt = jnp.concatenate([pos.astype(jnp.int32).T,
                            neg.astype(jnp.int32).T], axis=0)
    inpt = inp.astype(jnp.int32).T

    pred = _sc_pred(ctxt, inpt, emb, emb_f, emb_r)

    n_rows = B * KP // 128
    loss = pl.pallas_call(
        _bce_body,
        out_shape=jax.ShapeDtypeStruct((1, 1), jnp.float32),
    )(pred.reshape(n_rows, 128))
    return jnp.reshape(loss, ())


# speedup vs baseline: 11.1856x; 1.1006x over previous
import functools

import jax
import jax.numpy as jnp
from jax import lax
from jax.experimental import pallas as pl
from jax.experimental.pallas import tpu as pltpu
from jax.experimental.pallas import tpu_sc as plsc

B = 16384
D = 64
K = 28
KP = 32
NC = 2
NS = 16
NW = NC * NS
RW = B // NW
C = 4
NCH = RW // C
KC = C * K
NBUF = 4
NBLK = RW // 16


def _sc_body(ctxt_hbm, inpt_hbm, emb_hbm, embf_hbm, embr_hbm, pred_hbm,
             sta_ctx, sta_inp, idx_lin, e_lin, f_lin,
             r_a, r_b, r_c, r_d, e_a, e_b, e_c, e_d, f_a, f_b, f_c, f_d,
             pred_a, pred_b, pred_c, pred_d, sem_a, sem_b, sem_c, sem_d,
             psem_a, psem_b, psem_c, psem_d):
    wid = lax.axis_index("s") * NC + lax.axis_index("c")
    b0 = wid * RW
    hw = RW // 2

    lane = jnp.arange(16, dtype=jnp.int32)
    ctx_dst = lane * K
    e_dst = lane * 6
    f_dst = (lane // C) * 8 + lane % C

    pltpu.sync_copy(inpt_hbm.at[:, pl.ds(b0, RW)], sta_inp)

    for h in range(2):
        pltpu.sync_copy(ctxt_hbm.at[:, pl.ds(b0 + h * hw, hw)], sta_ctx)

        def tr_ctx(blk, carry):
            for k in range(K):
                vec = sta_ctx[k, pl.ds(blk * 16, 16)]
                plsc.store_scatter(
                    idx_lin,
                    [ctx_dst + ((h * (hw // 16) + blk) * (16 * K) + k)],
                    vec)
            return carry

        lax.fori_loop(0, hw // 16, tr_ctx, 0)

    def tr_e(blk, carry):
        for j in range(6):
            vec = sta_inp[1 + j, pl.ds(blk * 16, 16)]
            plsc.store_scatter(e_lin, [e_dst + (blk * 96 + j)], vec)
        vec = sta_inp[0, pl.ds(blk * 16, 16)]
        plsc.store_scatter(f_lin, [f_dst + (blk * 32)], vec)
        return carry

    lax.fori_loop(0, NBLK, tr_e, 0)

    def fire(c, r_buf, e_buf, f_buf, sem):
        pltpu.async_copy(embr_hbm.at[idx_lin.at[pl.ds(c * KC, KC)]],
                         r_buf, sem)
        pltpu.async_copy(embf_hbm.at[f_lin.at[pl.ds(c * 8, C)]], f_buf,
                         sem)
        pltpu.async_copy(emb_hbm.at[e_lin.at[pl.ds(c * 6 * C, 6 * C)]],
                         e_buf, sem)

    def drain(r_buf, e_buf, f_buf, sem):
        pltpu.make_async_copy(embr_hbm.at[pl.ds(0, KC)], r_buf,
                              sem).wait()
        pltpu.make_async_copy(embf_hbm.at[pl.ds(0, C)], f_buf, sem).wait()
        pltpu.make_async_copy(emb_hbm.at[pl.ds(0, 6 * C)], e_buf,
                              sem).wait()

    perms = [(lane + sh) % 16 for sh in (1, 2, 4, 8)]
    zeros = jnp.zeros((16,), jnp.float32)

    def compute(c, r_buf, e_buf, f_buf, pred_buf):
        def row_body(i, carry):
            f = [f_buf[i, pl.ds(s * 16, 16)] for s in range(8)]

            def e(j, s):
                return e_buf[i * 6 + j, pl.ds(s * 16, 16)]

            v = []
            for s in range(4):
                v.append((f[s] + e(0, s) + e(3, s)) * (1.0 / 3.0))
            for s in range(4):
                v.append((f[4 + s]
                          + (e(1, s) + e(2, s)) * 0.5
                          + (e(4, s) + e(5, s)) * 0.5) * (1.0 / 3.0))

            out = [zeros, zeros]
            for k in range(K):
                row = i * K + k
                p = [r_buf[row, pl.ds(s * 16, 16)] * v[s] for s in range(8)]
                while len(p) > 1:
                    p = [p[2 * t] + p[2 * t + 1] for t in range(len(p) // 2)]
                acc = p[0]
                for perm in perms:
                    acc = acc + acc.at[perm].get(mode="promise_in_bounds")
                half, ph = divmod(k, 16)
                out[half] = jnp.where(lane == ph, acc, out[half])
            pred_buf[pl.ds(i * KP, 16)] = out[0]
            pred_buf[pl.ds(i * KP + 16, 16)] = out[1]
            return carry

        lax.fori_loop(0, C, row_body, 0)

    def pred_fire(ch, pred_buf, psem):
        pltpu.async_copy(pred_buf, pred_hbm.at[pl.ds(ch * C * KP, C * KP)],
                         psem)

    def pred_drain(pred_buf, psem):
        pltpu.make_async_copy(pred_buf, pred_hbm.at[pl.ds(0, C * KP)],
                              psem).wait()

    bufs = [(r_a, e_a, f_a, sem_a, pred_a, psem_a),
            (r_b, e_b, f_b, sem_b, pred_b, psem_b),
            (r_c, e_c, f_c, sem_c, pred_c, psem_c),
            (r_d, e_d, f_d, sem_d, pred_d, psem_d)]

    for c in range(NBUF - 1):
        fire(c, bufs[c][0], bufs[c][1], bufs[c][2], bufs[c][3])

    def quad(j, carry):
        for u in range(NBUF):
            c = j * NBUF + u
            r_buf, e_buf, f_buf, sem, pred_buf, psem = bufs[u]
            rn, en, fn, semn, _, _ = bufs[(u + NBUF - 1) % NBUF]

            @pl.when(c + NBUF - 1 < NCH)
            def _():
                fire(c + NBUF - 1, rn, en, fn, semn)

            drain(r_buf, e_buf, f_buf, sem)

            @pl.when(c >= NBUF)
            def _():
                pred_drain(pred_buf, psem)

            compute(c, r_buf, e_buf, f_buf, pred_buf)
            pred_fire(wid * NCH + c, pred_buf, psem)
        return carry

    lax.fori_loop(0, NCH // NBUF, quad, 0)

    for u in range(NBUF):
        pred_drain(bufs[u][4], bufs[u][5])


_sc_pred = functools.partial(
    pl.kernel,
    out_type=jax.ShapeDtypeStruct((B * KP,), jnp.float32),
    mesh=plsc.VectorSubcoreMesh(core_axis_name="c", subcore_axis_name="s"),
    compiler_params=pltpu.CompilerParams(needs_layout_passes=False,
                                         use_tc_tiling_on_sc=False),
    scratch_types=(
        [
            pltpu.VMEM((K, RW // 2), jnp.int32),
            pltpu.VMEM((7, RW), jnp.int32),
            pltpu.VMEM((NCH * KC,), jnp.int32),
            pltpu.VMEM((NCH * 6 * C,), jnp.int32),
            pltpu.VMEM((NCH * 8,), jnp.int32),
        ]
        + [pltpu.VMEM((KC, 2 * D), jnp.float32)] * NBUF
        + [pltpu.VMEM((C * 6, D), jnp.float32)] * NBUF
        + [pltpu.VMEM((C, 2 * D), jnp.float32)] * NBUF
        + [pltpu.VMEM((C * KP,), jnp.float32)] * NBUF
        + [pltpu.SemaphoreType.DMA] * (2 * NBUF)
    ),
)(_sc_body)


def _bce_body(pred_ref, out_ref):
    x = pred_ref[...]
    rows = lax.broadcasted_iota(jnp.int32, x.shape, 0)
    cols = lax.broadcasted_iota(jnp.int32, x.shape, 1)
    k = (rows * x.shape[1] + cols) % KP
    label = k < 3
    valid = k < K
    p = jax.nn.sigmoid(x)
    p = jnp.clip(p, 1e-7, 1.0 - 1e-7)
    term = jnp.where(label, jnp.log(p), jnp.log(1.0 - p))
    term = jnp.where(valid, term, 0.0)
    loss = -jnp.sum(term) * (1.0 / (B * K))
    out_ref[...] = jnp.reshape(loss, (1, 1))


def kernel(inp, pos, neg, emb, emb_f, emb_r):
    ctxt = jnp.concatenate([pos.astype(jnp.int32).T,
                            neg.astype(jnp.int32).T], axis=0)
    inpt = inp.astype(jnp.int32).T

    pred = _sc_pred(ctxt, inpt, emb, emb_f, emb_r)

    n_rows = B * KP // 128
    loss = pl.pallas_call(
        _bce_body,
        out_shape=jax.ShapeDtypeStruct((1, 1), jnp.float32),
    )(pred.reshape(n_rows, 128))
    return jnp.reshape(loss, ())
